# baseline (speedup 1.0000x reference)
import jax
import jax.numpy as jnp
from jax import lax
from jax.experimental import pallas as pl
from jax.experimental.pallas import tpu as pltpu
from jax.experimental.pallas import tpu_sc as plsc
import numpy as np

B, S, V, H, P, T = 4, 2048, 100000, 1280, 2048, 2
EMBED_SCALE = float(np.sqrt(H))
LN_EPS = 1e-5

N = B * S
N_SC = 6144
N_TC = N - N_SC
NW = 32
TOK_PER_W = N_SC // NW
CHUNK = 16
NCHUNK = TOK_PER_W // CHUNK
TILE = 8
LANES = 16
HV = H // LANES
TB = 8


def _rsqrt_scalar(x):
    i = lax.bitcast_convert_type(x, jnp.int32)
    i = jnp.int32(0x5F3759DF) - (i >> 1)
    y = lax.bitcast_convert_type(i, jnp.float32)
    for _ in range(3):
        y = y * (1.5 - 0.5 * x * y * y)
    return y


def _sc_body(wid_hbm, pid_hbm, tid_hbm, word_hbm, pos_hbm, typ_hbm,
             gam_hbm, bet_hbm, out_hbm,
             widx, pidx, tidx, wrows0, wrows1, prows0, prows1,
             typv, gv, bv, semw0, semw1, semp0, semp1, semo0, semo1):
    w = lax.axis_index("s") * 2 + lax.axis_index("c")
    base = w * TOK_PER_W

    d_wi = pltpu.async_copy(wid_hbm.at[pl.ds(base, TOK_PER_W)], widx, semo0)
    d_pi = pltpu.async_copy(pid_hbm.at[pl.ds(base, TOK_PER_W)], pidx, semo0)
    d_ti = pltpu.async_copy(tid_hbm.at[pl.ds(base, TOK_PER_W)], tidx, semo1)
    d_ty = pltpu.async_copy(typ_hbm, typv, semo1)
    d_g = pltpu.async_copy(gam_hbm, gv, semo1)
    d_b = pltpu.async_copy(bet_hbm, bv, semo1)
    d_wi.wait()
    d_pi.wait()

    wbufs = (wrows0, wrows1)
    pbufs = (prows0, prows1)
    semws = (semw0, semw1)
    semps = (semp0, semp1)
    semos = (semo0, semo1)

    def gather_descr(c, b):
        dw = pltpu.make_async_copy(
            word_hbm.at[widx.at[pl.ds(c * CHUNK, CHUNK)]], wbufs[b], semws[b])
        dp = pltpu.make_async_copy(
            pos_hbm.at[pidx.at[pl.ds(c * CHUNK, CHUNK)]], pbufs[b], semps[b])
        return dw, dp

    def out_descr(c, b):
        return pltpu.make_async_copy(
            wbufs[b], out_hbm.at[pl.ds(base + c * CHUNK, CHUNK)], semos[b])

    def compute_chunk(c, b):
        wr = wbufs[b]
        pr = pbufs[b]
        tgrp = tidx[pl.ds(c * CHUNK, LANES)]
        lanes_iota = lax.iota(jnp.int32, LANES)
        means = []
        rstds = []
        for t in range(CHUNK // TILE):
            i0 = t * TILE
            tf = []
            for k in range(TILE):
                m = lanes_iota == (i0 + k)
                tf.append(jnp.sum(jnp.where(m, tgrp, 0)).astype(jnp.float32))

            zero = jnp.zeros((LANES,), jnp.float32)

            def p1(jj, carry):
                s1 = list(carry[:TILE])
                s2 = list(carry[TILE:])
                sl = pl.ds(jj * LANES, LANES)
                t0 = typv[0, sl]
                dt = typv[1, sl] - t0
                for k in range(TILE):
                    y = (wr[i0 + k, sl] * EMBED_SCALE
                         + pr[i0 + k, sl] + (t0 + tf[k] * dt))
                    wr[i0 + k, sl] = y
                    s1[k] = s1[k] + y
                    s2[k] = s2[k] + y * y
                return tuple(s1) + tuple(s2)

            accs = lax.fori_loop(0, HV, p1, (zero,) * (2 * TILE))

            for k in range(TILE):
                mean = jnp.sum(accs[k]) * (1.0 / H)
                msq = jnp.sum(accs[TILE + k]) * (1.0 / H)
                var = msq - mean * mean
                rstds.append(_rsqrt_scalar(var + LN_EPS))
                means.append(mean)

        def p2(jj, carry):
            sl = pl.ds(jj * LANES, LANES)
            g = gv[sl]
            bb = bv[sl]
            for k in range(CHUNK):
                y = wr[k, sl]
                wr[k, sl] = (y - means[k]) * rstds[k] * g + bb
            return carry

        lax.fori_loop(0, HV, p2, 0)

    def start_gather(c, b):
        for d in gather_descr(c, b):
            d.start()

    def wait_gather(c, b):
        for d in gather_descr(c, b):
            d.wait()

    start_gather(0, 0)
    d_ti.wait()
    d_ty.wait()
    d_g.wait()
    d_b.wait()
    wait_gather(0, 0)
    start_gather(1, 1)
    compute_chunk(0, 0)
    out_descr(0, 0).start()

    def pair_body(p, _):
        for b, coff in ((1, 1), (0, 2)):
            c = 2 * p + coff
            wait_gather(c, b)
            out_descr(c - 1, 1 - b).wait()
            start_gather(c + 1, 1 - b)
            compute_chunk(c, b)
            out_descr(c, b).start()
        return 0

    lax.fori_loop(0, (NCHUNK - 2) // 2, pair_body, 0)

    wait_gather(NCHUNK - 1, 1)
    out_descr(NCHUNK - 2, 0).wait()
    compute_chunk(NCHUNK - 1, 1)
    out_descr(NCHUNK - 1, 1).start()
    out_descr(NCHUNK - 1, 1).wait()


def _tc_body(widp, pidp, tidp, *refs):
    del widp, pidp
    wrefs = refs[:TB]
    prefs = refs[TB:2 * TB]
    typ_ref, g_ref, b_ref, out_ref = refs[2 * TB:]
    i = pl.program_id(0)
    t0 = typ_ref[0:1, :]
    t1 = typ_ref[1:2, :]
    rows = []
    for k in range(TB):
        tid = tidp[i * TB + k]
        trow = jnp.where(tid == 0, t0, t1)
        rows.append(wrefs[k][0] * EMBED_SCALE + prefs[k][0] + trow)
    ys = jnp.concatenate(rows, axis=0)
    mean = jnp.mean(ys, axis=-1, keepdims=True)
    var = jnp.mean(jnp.square(ys - mean), axis=-1, keepdims=True)
    normed = (ys - mean) * jax.lax.rsqrt(var + LN_EPS)
    out_ref[...] = normed * g_ref[...] + b_ref[...]


def _tc_call(wid, pid, tid, word, pos, typ, gamma, beta):
    def wmap(k):
        return lambda i, W, P, T: (W[i * TB + k], 0, 0)

    def pmap(k):
        return lambda i, W, P, T: (P[i * TB + k], 0, 0)

    grid_spec = pltpu.PrefetchScalarGridSpec(
        num_scalar_prefetch=3,
        grid=(N_TC // TB,),
        in_specs=(
            [pl.BlockSpec((1, 1, H), wmap(k)) for k in range(TB)]
            + [pl.BlockSpec((1, 1, H), pmap(k)) for k in range(TB)]
            + [pl.BlockSpec((T, H), lambda i, W, P, T_: (0, 0)),
               pl.BlockSpec((1, H), lambda i, W, P, T_: (0, 0)),
               pl.BlockSpec((1, H), lambda i, W, P, T_: (0, 0))]
        ),
        out_specs=pl.BlockSpec((TB, H), lambda i, W, P, T_: (i, 0)),
    )
    call = pl.pallas_call(
        _tc_body,
        grid_spec=grid_spec,
        out_shape=jax.ShapeDtypeStruct((N_TC, H), jnp.float32),
    )
    word3 = word.reshape(V, 1, H)
    pos3 = pos.reshape(P, 1, H)
    return call(wid, pid, tid,
                *([word3] * TB), *([pos3] * TB),
                typ, gamma.reshape(1, H), beta.reshape(1, H))


@jax.jit
def _embeddings_sc(wid, pid, tid, word, pos, typ, gamma, beta):
    mesh = plsc.VectorSubcoreMesh(core_axis_name="c", subcore_axis_name="s")
    call = pl.kernel(
        _sc_body,
        out_type=jax.ShapeDtypeStruct((N_SC, H), jnp.float32),
        mesh=mesh,
        compiler_params=pltpu.CompilerParams(needs_layout_passes=False),
        scratch_types=[
            pltpu.VMEM((TOK_PER_W,), jnp.int32),
            pltpu.VMEM((TOK_PER_W,), jnp.int32),
            pltpu.VMEM((TOK_PER_W,), jnp.int32),
            pltpu.VMEM((CHUNK, H), jnp.float32),
            pltpu.VMEM((CHUNK, H), jnp.float32),
            pltpu.VMEM((CHUNK, H), jnp.float32),
            pltpu.VMEM((CHUNK, H), jnp.float32),
            pltpu.VMEM((T, H), jnp.float32),
            pltpu.VMEM((H,), jnp.float32),
            pltpu.VMEM((H,), jnp.float32),
            pltpu.SemaphoreType.DMA,
            pltpu.SemaphoreType.DMA,
            pltpu.SemaphoreType.DMA,
            pltpu.SemaphoreType.DMA,
            pltpu.SemaphoreType.DMA,
            pltpu.SemaphoreType.DMA,
        ],
    )
    return call(wid, pid, tid, word, pos, typ, gamma, beta)


@jax.jit
def _embeddings(wid, pid, tid, word, pos, typ, gamma, beta):
    out_sc = _embeddings_sc(wid[:N_SC], pid[:N_SC], tid[:N_SC],
                            word, pos, typ, gamma, beta)
    out_tc = _tc_call(wid[N_SC:], pid[N_SC:], tid[N_SC:],
                      word, pos, typ, gamma, beta)
    return jnp.concatenate([out_sc, out_tc], axis=0)


def kernel(input_ids, position_ids, type_token_ids, word_embedding,
           position_embedding, type_embedding, gamma, beta):
    wid = input_ids.reshape(N).astype(jnp.int32)
    pid = position_ids.reshape(N).astype(jnp.int32)
    tid = type_token_ids.reshape(N).astype(jnp.int32)
    out = _embeddings(wid, pid, tid, word_embedding, position_embedding,
                      type_embedding, gamma, beta)
    return out.reshape(B, S, H)


# speedup vs baseline: 8.7490x; 8.7490x over previous
import jax
import jax.numpy as jnp
from jax import lax
from jax.experimental import pallas as pl
from jax.experimental.pallas import tpu as pltpu
from jax.experimental.pallas import tpu_sc as plsc
import numpy as np

B, S, V, H, P, T = 4, 2048, 100000, 1280, 2048, 2
EMBED_SCALE = float(np.sqrt(H))
LN_EPS = 1e-5

N = B * S
NW = 32
TOK_PER_W = N // NW
CHUNK = 16
NCHUNK = TOK_PER_W // CHUNK
TILE = 8
LANES = 16
HV = H // LANES


def _rsqrt_scalar(x):
    i = lax.bitcast_convert_type(x, jnp.int32)
    i = jnp.int32(0x5F3759DF) - (i >> 1)
    y = lax.bitcast_convert_type(i, jnp.float32)
    for _ in range(3):
        y = y * (1.5 - 0.5 * x * y * y)
    return y


def _sc_body(wid_hbm, pid_hbm, tid_hbm, word_hbm, pos_hbm, typ_hbm,
             gam_hbm, bet_hbm, out_hbm,
             widx, pidx, tidx, wrows0, wrows1, prows0, prows1,
             typv, gv, bv, semw0, semw1, semp0, semp1, semo0, semo1):
    w = lax.axis_index("s") * 2 + lax.axis_index("c")
    base = w * TOK_PER_W

    d_wi = pltpu.async_copy(wid_hbm.at[pl.ds(base, TOK_PER_W)], widx, semo0)
    d_pi = pltpu.async_copy(pid_hbm.at[pl.ds(base, TOK_PER_W)], pidx, semo0)
    d_ti = pltpu.async_copy(tid_hbm.at[pl.ds(base, TOK_PER_W)], tidx, semo1)
    d_ty = pltpu.async_copy(typ_hbm, typv, semo1)
    d_g = pltpu.async_copy(gam_hbm, gv, semo1)
    d_b = pltpu.async_copy(bet_hbm, bv, semo1)
    d_wi.wait()
    d_pi.wait()

    wbufs = (wrows0, wrows1)
    pbufs = (prows0, prows1)
    semws = (semw0, semw1)
    semps = (semp0, semp1)
    semos = (semo0, semo1)

    def gather_descr(c, b):
        dw = pltpu.make_async_copy(
            word_hbm.at[widx.at[pl.ds(c * CHUNK, CHUNK)]], wbufs[b], semws[b])
        dp = pltpu.make_async_copy(
            pos_hbm.at[pidx.at[pl.ds(c * CHUNK, CHUNK)]], pbufs[b], semps[b])
        return dw, dp

    def out_descr(c, b):
        return pltpu.make_async_copy(
            wbufs[b], out_hbm.at[pl.ds(base + c * CHUNK, CHUNK)], semos[b])

    def compute_chunk(c, b):
        wr = wbufs[b]
        pr = pbufs[b]
        tgrp = tidx[pl.ds(c * CHUNK, LANES)]
        lanes_iota = lax.iota(jnp.int32, LANES)
        means = []
        rstds = []
        for t in range(CHUNK // TILE):
            i0 = t * TILE
            tf = []
            for k in range(TILE):
                m = lanes_iota == (i0 + k)
                tf.append(jnp.sum(jnp.where(m, tgrp, 0)).astype(jnp.float32))

            zero = jnp.zeros((LANES,), jnp.float32)

            def p1(jj, carry):
                s1 = list(carry[:TILE])
                s2 = list(carry[TILE:])
                sl = pl.ds(jj * LANES, LANES)
                t0 = typv[0, sl]
                dt = typv[1, sl] - t0
                for k in range(TILE):
                    y = (wr[i0 + k, sl] * EMBED_SCALE
                         + pr[i0 + k, sl] + (t0 + tf[k] * dt))
                    wr[i0 + k, sl] = y
                    s1[k] = s1[k] + y
                    s2[k] = s2[k] + y * y
                return tuple(s1) + tuple(s2)

            accs = lax.fori_loop(0, HV, p1, (zero,) * (2 * TILE))

            for k in range(TILE):
                mean = jnp.sum(accs[k]) * (1.0 / H)
                msq = jnp.sum(accs[TILE + k]) * (1.0 / H)
                var = msq - mean * mean
                rstds.append(_rsqrt_scalar(var + LN_EPS))
                means.append(mean)

        def p2(jj, carry):
            sl = pl.ds(jj * LANES, LANES)
            g = gv[sl]
            bb = bv[sl]
            for k in range(CHUNK):
                y = wr[k, sl]
                wr[k, sl] = (y - means[k]) * rstds[k] * g + bb
            return carry

        lax.fori_loop(0, HV, p2, 0)

    def start_gather(c, b):
        for d in gather_descr(c, b):
            d.start()

    def wait_gather(c, b):
        for d in gather_descr(c, b):
            d.wait()

    start_gather(0, 0)
    d_ti.wait()
    d_ty.wait()
    d_g.wait()
    d_b.wait()
    wait_gather(0, 0)
    start_gather(1, 1)
    compute_chunk(0, 0)
    out_descr(0, 0).start()

    def pair_body(p, _):
        for b, coff in ((1, 1), (0, 2)):
            c = 2 * p + coff
            wait_gather(c, b)
            out_descr(c - 1, 1 - b).wait()
            start_gather(c + 1, 1 - b)
            compute_chunk(c, b)
            out_descr(c, b).start()
        return 0

    lax.fori_loop(0, (NCHUNK - 2) // 2, pair_body, 0)

    wait_gather(NCHUNK - 1, 1)
    out_descr(NCHUNK - 2, 0).wait()
    compute_chunk(NCHUNK - 1, 1)
    out_descr(NCHUNK - 1, 1).start()
    out_descr(NCHUNK - 1, 1).wait()


@jax.jit
def _embeddings_sc(wid, pid, tid, word, pos, typ, gamma, beta):
    mesh = plsc.VectorSubcoreMesh(core_axis_name="c", subcore_axis_name="s")
    call = pl.kernel(
        _sc_body,
        out_type=jax.ShapeDtypeStruct((N, H), jnp.float32),
        mesh=mesh,
        compiler_params=pltpu.CompilerParams(needs_layout_passes=False),
        scratch_types=[
            pltpu.VMEM((TOK_PER_W,), jnp.int32),
            pltpu.VMEM((TOK_PER_W,), jnp.int32),
            pltpu.VMEM((TOK_PER_W,), jnp.int32),
            pltpu.VMEM((CHUNK, H), jnp.float32),
            pltpu.VMEM((CHUNK, H), jnp.float32),
            pltpu.VMEM((CHUNK, H), jnp.float32),
            pltpu.VMEM((CHUNK, H), jnp.float32),
            pltpu.VMEM((T, H), jnp.float32),
            pltpu.VMEM((H,), jnp.float32),
            pltpu.VMEM((H,), jnp.float32),
            pltpu.SemaphoreType.DMA,
            pltpu.SemaphoreType.DMA,
            pltpu.SemaphoreType.DMA,
            pltpu.SemaphoreType.DMA,
            pltpu.SemaphoreType.DMA,
            pltpu.SemaphoreType.DMA,
        ],
    )
    return call(wid, pid, tid, word, pos, typ, gamma, beta)


def kernel(input_ids, position_ids, type_token_ids, word_embedding,
           position_embedding, type_embedding, gamma, beta):
    wid = input_ids.reshape(N).astype(jnp.int32)
    pid = position_ids.reshape(N).astype(jnp.int32)
    tid = type_token_ids.reshape(N).astype(jnp.int32)
    out = _embeddings_sc(wid, pid, tid, word_embedding, position_embedding,
                         type_embedding, gamma, beta)
    return out.reshape(B, S, H)
